# Initial kernel scaffold; baseline (speedup 1.0000x reference)
import jax
import jax.numpy as jnp
from jax.experimental import pallas as pl


def kernel(x, edge_index, W0, b0, W1, b1, W2, b2):
    raise NotImplementedError("write your pallas kernel here")


# speedup vs baseline: 6.9366x; 6.9366x over previous
import functools

import jax
import jax.numpy as jnp
from jax import lax
from jax.experimental import pallas as pl
from jax.experimental.pallas import tpu as pltpu
from jax.experimental.pallas import tpu_sc as plsc

N_NODES = 10000
N_EDGES = 320000
D = 128

NC = 2
NS = 16
NW = NC * NS
LANES = 16

CHUNK = 128
NCH = -(-N_EDGES // (NW * CHUNK))
NCH += NCH % 2
E_PAD = NW * NCH * CHUNK

NP = 10112
RPT = NP // NS
DW = 16

BLK = 2528
G = NP // BLK


def _zero_fill(buf, rows, width):
    zeros = jnp.zeros((LANES,), jnp.float32)

    def row(i, c):
        for j in range(width // LANES):
            buf[i, pl.ds(j * LANES, LANES)] = zeros
        return c

    lax.fori_loop(0, rows, row, 0)


def _fill_ones(buf, rows, width):
    ones = jnp.ones((LANES,), jnp.float32)

    def row(i, c):
        for j in range(width // LANES):
            buf[i, pl.ds(j * LANES, LANES)] = ones
        return c

    lax.fori_loop(0, rows, row, 0)


def _zero_spmem_rows(zbuf, sp, r0, width):
    full = RPT // CHUNK
    for k in range(full):
        pltpu.sync_copy(zbuf, sp.at[pl.ds(r0 + k * CHUNK, CHUNK)])
    rem = RPT - full * CHUNK
    if rem:
        pltpu.sync_copy(zbuf.at[pl.ds(0, rem)], sp.at[pl.ds(r0 + full * CHUNK, rem)])


def _agg_body(table, src_hbm, dst_hbm, out, src_v, dst_v, buf, z_sp):
    cid = lax.axis_index("c")
    sid = lax.axis_index("s")
    wid = cid * NS + sid
    pltpu.sync_copy(src_hbm.at[wid], src_v)
    pltpu.sync_copy(dst_hbm.at[wid], dst_v)
    _zero_fill(buf, CHUNK, D)
    r0 = sid * RPT
    _zero_spmem_rows(buf, z_sp, r0, D)
    plsc.subcore_barrier()

    def body(j, c):
        pltpu.sync_copy(table.at[src_v.at[j]], buf)
        pltpu.sync_copy(buf, z_sp.at[dst_v.at[j]], add=True)
        return c

    lax.fori_loop(0, NCH, body, 0)
    plsc.subcore_barrier()
    pltpu.sync_copy(z_sp.at[pl.ds(r0, RPT)], out.at[cid].at[pl.ds(r0, RPT)])


@functools.lru_cache(maxsize=None)
def _agg_kernel():
    return pl.kernel(
        _agg_body,
        out_type=jax.ShapeDtypeStruct((NC, NP, D), jnp.float32),
        mesh=plsc.VectorSubcoreMesh(core_axis_name="c", subcore_axis_name="s",
                                    num_cores=NC, num_subcores=NS),
        scratch_types=[
            pltpu.VMEM((NCH, CHUNK), jnp.int32),
            pltpu.VMEM((NCH, CHUNK), jnp.int32),
            pltpu.VMEM((CHUNK, D), jnp.float32),
            pltpu.VMEM_SHARED((NP, D), jnp.float32),
        ],
    )


def _agg(table, src_s, dst_s):
    return _agg_kernel()(table, src_s, dst_s)


def _deg_body(dst_hbm, out, dst_v, obuf, zbuf, d_sp):
    cid = lax.axis_index("c")
    sid = lax.axis_index("s")
    wid = cid * NS + sid
    pltpu.sync_copy(dst_hbm.at[wid], dst_v)
    _fill_ones(obuf, CHUNK, DW)
    _zero_fill(zbuf, CHUNK, DW)
    r0 = sid * RPT
    _zero_spmem_rows(zbuf, d_sp, r0, DW)
    plsc.subcore_barrier()

    def body(j, c):
        pltpu.sync_copy(obuf, d_sp.at[dst_v.at[j]], add=True)
        return c

    lax.fori_loop(0, NCH, body, 0)
    plsc.subcore_barrier()
    pltpu.sync_copy(d_sp.at[pl.ds(r0, RPT)], out.at[cid].at[pl.ds(r0, RPT)])


@functools.lru_cache(maxsize=None)
def _deg_kernel():
    return pl.kernel(
        _deg_body,
        out_type=jax.ShapeDtypeStruct((NC, NP, DW), jnp.float32),
        mesh=plsc.VectorSubcoreMesh(core_axis_name="c", subcore_axis_name="s",
                                    num_cores=NC, num_subcores=NS),
        scratch_types=[
            pltpu.VMEM((NCH, CHUNK), jnp.int32),
            pltpu.VMEM((CHUNK, DW), jnp.float32),
            pltpu.VMEM((CHUNK, DW), jnp.float32),
            pltpu.VMEM_SHARED((NP, DW), jnp.float32),
        ],
    )


def _deg(dst_s):
    return _deg_kernel()(dst_s)


def _tc_first_body(x_ref, d_ref, w_ref, dis_ref, y_ref):
    d = d_ref[...]
    deg = d[0, :, 0:1] + d[1, :, 0:1] + 1.0
    dis = jnp.broadcast_to(lax.rsqrt(deg), (BLK, D))
    dis_ref[...] = dis
    y_ref[...] = dis * jnp.dot(x_ref[...], w_ref[...],
                               preferred_element_type=jnp.float32)


def _tc_first(xp, deg, w0):
    return pl.pallas_call(
        _tc_first_body,
        grid=(G,),
        in_specs=[
            pl.BlockSpec((BLK, D), lambda i: (i, 0)),
            pl.BlockSpec((NC, BLK, DW), lambda i: (0, i, 0)),
            pl.BlockSpec((D, D), lambda i: (0, 0)),
        ],
        out_specs=[
            pl.BlockSpec((BLK, D), lambda i: (i, 0)),
            pl.BlockSpec((BLK, D), lambda i: (i, 0)),
        ],
        out_shape=[
            jax.ShapeDtypeStruct((NP, D), jnp.float32),
            jax.ShapeDtypeStruct((NP, D), jnp.float32),
        ],
    )(xp, deg, w0)


def _elu(h):
    return jnp.where(h > 0, h, jnp.exp(jnp.minimum(h, 0.0)) - 1.0)


def _tc_mid_body(dis_ref, z_ref, y_ref, b_ref, w_ref, out_ref):
    z = z_ref[...]
    h = dis_ref[...] * (z[0] + z[1] + y_ref[...]) + b_ref[...]
    h = _elu(h)
    out_ref[...] = dis_ref[...] * jnp.dot(h, w_ref[...],
                                          preferred_element_type=jnp.float32)


def _tc_mid(dis, z, y, b, w_next):
    return pl.pallas_call(
        _tc_mid_body,
        grid=(G,),
        in_specs=[
            pl.BlockSpec((BLK, D), lambda i: (i, 0)),
            pl.BlockSpec((NC, BLK, D), lambda i: (0, i, 0)),
            pl.BlockSpec((BLK, D), lambda i: (i, 0)),
            pl.BlockSpec((1, D), lambda i: (0, 0)),
            pl.BlockSpec((D, D), lambda i: (0, 0)),
        ],
        out_specs=pl.BlockSpec((BLK, D), lambda i: (i, 0)),
        out_shape=jax.ShapeDtypeStruct((NP, D), jnp.float32),
    )(dis, z, y, b, w_next)


def _tc_last_body(dis_ref, z_ref, y_ref, b_ref, out_ref):
    z = z_ref[...]
    h = dis_ref[...] * (z[0] + z[1] + y_ref[...]) + b_ref[...]
    out_ref[...] = _elu(h)


def _tc_last(dis, z, y, b):
    return pl.pallas_call(
        _tc_last_body,
        grid=(G,),
        in_specs=[
            pl.BlockSpec((BLK, D), lambda i: (i, 0)),
            pl.BlockSpec((NC, BLK, D), lambda i: (0, i, 0)),
            pl.BlockSpec((BLK, D), lambda i: (i, 0)),
            pl.BlockSpec((1, D), lambda i: (0, 0)),
        ],
        out_specs=pl.BlockSpec((BLK, D), lambda i: (i, 0)),
        out_shape=jax.ShapeDtypeStruct((NP, D), jnp.float32),
    )(dis, z, y, b)


def kernel(x, edge_index, W0, b0, W1, b1, W2, b2):
    ei = edge_index.astype(jnp.int32)
    padlen = E_PAD - N_EDGES
    src_s = jnp.concatenate(
        [ei[0], jnp.zeros((padlen,), jnp.int32)]).reshape(NW, NCH, CHUNK)
    dst_s = jnp.concatenate(
        [ei[1], jnp.full((padlen,), N_NODES, jnp.int32)]).reshape(NW, NCH, CHUNK)
    xp = jnp.pad(x, ((0, NP - N_NODES), (0, 0)))

    deg = _deg(dst_s)
    dis, y = _tc_first(xp, deg, W0)
    for b, w_next in ((b0, W1), (b1, W2)):
        z = _agg(y, src_s, dst_s)
        y = _tc_mid(dis, z, y, b.reshape(1, D), w_next)
    z = _agg(y, src_s, dst_s)
    out = _tc_last(dis, z, y, b2.reshape(1, D))
    return out[:N_NODES]


# speedup vs baseline: 8.5438x; 1.2317x over previous
import functools

import jax
import jax.numpy as jnp
from jax import lax
from jax.experimental import pallas as pl
from jax.experimental.pallas import tpu as pltpu
from jax.experimental.pallas import tpu_sc as plsc

N_NODES = 10000
N_EDGES = 320000
D = 128

NC = 2
NS = 16
NW = NC * NS
LANES = 16

CHUNK = 128
NCH = -(-N_EDGES // (NW * CHUNK))
NCH += NCH % 2
E_PAD = NW * NCH * CHUNK

NP = 10112
RPT = NP // NS
DW = 16

BLK = 2528
G = NP // BLK


def _zero_fill(buf, rows, width):
    zeros = jnp.zeros((LANES,), jnp.float32)

    def row(i, c):
        for j in range(width // LANES):
            buf[i, pl.ds(j * LANES, LANES)] = zeros
        return c

    lax.fori_loop(0, rows, row, 0)


def _fill_ones(buf, rows, width):
    ones = jnp.ones((LANES,), jnp.float32)

    def row(i, c):
        for j in range(width // LANES):
            buf[i, pl.ds(j * LANES, LANES)] = ones
        return c

    lax.fori_loop(0, rows, row, 0)


def _zero_spmem_rows(zbuf, sp, r0, width):
    full = RPT // CHUNK
    for k in range(full):
        pltpu.sync_copy(zbuf, sp.at[pl.ds(r0 + k * CHUNK, CHUNK)])
    rem = RPT - full * CHUNK
    if rem:
        pltpu.sync_copy(zbuf.at[pl.ds(0, rem)], sp.at[pl.ds(r0 + full * CHUNK, rem)])


NBUF = 3


def _agg_body(table, idx_hbm, out, idxv, bufs, z_sp, *sems):
    si = sems[:NBUF]
    sg = sems[NBUF:2 * NBUF]
    ss = sems[2 * NBUF:]
    cid = lax.axis_index("c")
    sid = lax.axis_index("s")
    wid = cid * NS + sid
    _zero_fill(bufs.at[0], CHUNK, D)
    r0 = sid * RPT
    _zero_spmem_rows(bufs.at[0], z_sp, r0, D)
    plsc.subcore_barrier()

    hi = [None] * NCH
    hg = [None] * NCH
    hs = [None] * NCH
    for j in range(NCH + 2):
        if j < NCH:
            b = j % NBUF
            if j >= NBUF:
                hs[j - NBUF].wait()
            hi[j] = pltpu.async_copy(idx_hbm.at[wid].at[j],
                                     idxv.at[b], si[b])
        if 1 <= j < NCH + 1:
            p = j - 1
            pb = p % NBUF
            hi[p].wait()
            hg[p] = pltpu.async_copy(table.at[idxv.at[pb].at[0]],
                                     bufs.at[pb], sg[pb])
        if j >= 2:
            q = j - 2
            qb = q % NBUF
            hg[q].wait()
            hs[q] = pltpu.async_copy(bufs.at[qb], z_sp.at[idxv.at[qb].at[1]],
                                     ss[qb], add=True)
    for j in range(max(0, NCH - NBUF), NCH):
        hs[j].wait()
    plsc.subcore_barrier()
    pltpu.sync_copy(z_sp.at[pl.ds(r0, RPT)], out.at[cid].at[pl.ds(r0, RPT)])


@functools.lru_cache(maxsize=None)
def _agg_kernel():
    return pl.kernel(
        _agg_body,
        out_type=jax.ShapeDtypeStruct((NC, NP, D), jnp.float32),
        mesh=plsc.VectorSubcoreMesh(core_axis_name="c", subcore_axis_name="s",
                                    num_cores=NC, num_subcores=NS),
        scratch_types=[
            pltpu.VMEM((NBUF, 2, CHUNK), jnp.int32),
            pltpu.VMEM((NBUF, CHUNK, D), jnp.float32),
            pltpu.VMEM_SHARED((NP, D), jnp.float32),
        ] + [pltpu.SemaphoreType.DMA] * (3 * NBUF),
    )


def _agg(table, idx_s):
    return _agg_kernel()(table, idx_s)


def _deg_body(dst_hbm, out, dst_v, obuf, zbuf, d_sp):
    cid = lax.axis_index("c")
    sid = lax.axis_index("s")
    wid = cid * NS + sid
    pltpu.sync_copy(dst_hbm.at[wid], dst_v)
    _fill_ones(obuf, CHUNK, DW)
    _zero_fill(zbuf, CHUNK, DW)
    r0 = sid * RPT
    _zero_spmem_rows(zbuf, d_sp, r0, DW)
    plsc.subcore_barrier()

    def body(j, c):
        pltpu.sync_copy(obuf, d_sp.at[dst_v.at[j]], add=True)
        return c

    lax.fori_loop(0, NCH, body, 0)
    plsc.subcore_barrier()
    pltpu.sync_copy(d_sp.at[pl.ds(r0, RPT)], out.at[cid].at[pl.ds(r0, RPT)])


@functools.lru_cache(maxsize=None)
def _deg_kernel():
    return pl.kernel(
        _deg_body,
        out_type=jax.ShapeDtypeStruct((NC, NP, DW), jnp.float32),
        mesh=plsc.VectorSubcoreMesh(core_axis_name="c", subcore_axis_name="s",
                                    num_cores=NC, num_subcores=NS),
        scratch_types=[
            pltpu.VMEM((NCH, CHUNK), jnp.int32),
            pltpu.VMEM((CHUNK, DW), jnp.float32),
            pltpu.VMEM((CHUNK, DW), jnp.float32),
            pltpu.VMEM_SHARED((NP, DW), jnp.float32),
        ],
    )


def _deg(dst_s):
    return _deg_kernel()(dst_s)


def _tc_first_body(x_ref, d_ref, w_ref, dis_ref, y_ref):
    d = d_ref[...]
    deg = d[0, :, 0:1] + d[1, :, 0:1] + 1.0
    dis = jnp.broadcast_to(lax.rsqrt(deg), (BLK, D))
    dis_ref[...] = dis
    y_ref[...] = dis * jnp.dot(x_ref[...], w_ref[...],
                               preferred_element_type=jnp.float32)


def _tc_first(xp, deg, w0):
    return pl.pallas_call(
        _tc_first_body,
        grid=(G,),
        in_specs=[
            pl.BlockSpec((BLK, D), lambda i: (i, 0)),
            pl.BlockSpec((NC, BLK, DW), lambda i: (0, i, 0)),
            pl.BlockSpec((D, D), lambda i: (0, 0)),
        ],
        out_specs=[
            pl.BlockSpec((BLK, D), lambda i: (i, 0)),
            pl.BlockSpec((BLK, D), lambda i: (i, 0)),
        ],
        out_shape=[
            jax.ShapeDtypeStruct((NP, D), jnp.float32),
            jax.ShapeDtypeStruct((NP, D), jnp.float32),
        ],
    )(xp, deg, w0)


def _elu(h):
    return jnp.where(h > 0, h, jnp.exp(jnp.minimum(h, 0.0)) - 1.0)


def _tc_mid_body(dis_ref, z_ref, y_ref, b_ref, w_ref, out_ref):
    z = z_ref[...]
    h = dis_ref[...] * (z[0] + z[1] + y_ref[...]) + b_ref[...]
    h = _elu(h)
    out_ref[...] = dis_ref[...] * jnp.dot(h, w_ref[...],
                                          preferred_element_type=jnp.float32)


def _tc_mid(dis, z, y, b, w_next):
    return pl.pallas_call(
        _tc_mid_body,
        grid=(G,),
        in_specs=[
            pl.BlockSpec((BLK, D), lambda i: (i, 0)),
            pl.BlockSpec((NC, BLK, D), lambda i: (0, i, 0)),
            pl.BlockSpec((BLK, D), lambda i: (i, 0)),
            pl.BlockSpec((1, D), lambda i: (0, 0)),
            pl.BlockSpec((D, D), lambda i: (0, 0)),
        ],
        out_specs=pl.BlockSpec((BLK, D), lambda i: (i, 0)),
        out_shape=jax.ShapeDtypeStruct((NP, D), jnp.float32),
    )(dis, z, y, b, w_next)


def _tc_last_body(dis_ref, z_ref, y_ref, b_ref, out_ref):
    z = z_ref[...]
    h = dis_ref[...] * (z[0] + z[1] + y_ref[...]) + b_ref[...]
    out_ref[...] = _elu(h)


def _tc_last(dis, z, y, b):
    return pl.pallas_call(
        _tc_last_body,
        grid=(G,),
        in_specs=[
            pl.BlockSpec((BLK, D), lambda i: (i, 0)),
            pl.BlockSpec((NC, BLK, D), lambda i: (0, i, 0)),
            pl.BlockSpec((BLK, D), lambda i: (i, 0)),
            pl.BlockSpec((1, D), lambda i: (0, 0)),
        ],
        out_specs=pl.BlockSpec((BLK, D), lambda i: (i, 0)),
        out_shape=jax.ShapeDtypeStruct((NP, D), jnp.float32),
    )(dis, z, y, b)


def kernel(x, edge_index, W0, b0, W1, b1, W2, b2):
    ei = edge_index.astype(jnp.int32)
    padlen = E_PAD - N_EDGES
    src_s = jnp.concatenate(
        [ei[0], jnp.zeros((padlen,), jnp.int32)]).reshape(NW, NCH, CHUNK)
    dst_s = jnp.concatenate(
        [ei[1], jnp.full((padlen,), N_NODES, jnp.int32)]).reshape(NW, NCH, CHUNK)
    idx_s = jnp.stack([src_s, dst_s], axis=2)
    xp = jnp.pad(x, ((0, NP - N_NODES), (0, 0)))

    deg = _deg(dst_s)
    dis, y = _tc_first(xp, deg, W0)
    for b, w_next in ((b0, W1), (b1, W2)):
        z = _agg(y, idx_s)
        y = _tc_mid(dis, z, y, b.reshape(1, D), w_next)
    z = _agg(y, idx_s)
    out = _tc_last(dis, z, y, b2.reshape(1, D))
    return out[:N_NODES]
